# Initial kernel scaffold; baseline (speedup 1.0000x reference)
import jax
import jax.numpy as jnp
from jax.experimental import pallas as pl


def kernel(x, categories):
    raise NotImplementedError("write your pallas kernel here")


# speedup vs baseline: 7.2166x; 7.2166x over previous
import jax
import jax.numpy as jnp
from jax.experimental import pallas as pl

VIGILANCE = 0.5
BLOCK_ROWS = 64


def _fuzzy_art_block(x_ref, ct_ref, out_ref, idx_ref):
    bi = x_ref.shape[0]
    d = ct_ref.shape[0]

    def row_body(s, carry):
        xrow = x_ref[pl.ds(s, 1), :]
        xcol = xrow.reshape(d, 1)
        mins = jnp.minimum(ct_ref[...], xcol)
        sums = jnp.sum(mins, axis=0, keepdims=True)
        out_ref[pl.ds(s, 1), :] = sums
        return carry

    jax.lax.fori_loop(0, bi, row_body, 0)

    xsum = jnp.sum(x_ref[...], axis=1, keepdims=True)
    match = out_ref[...] / xsum
    thr = jnp.where(match >= VIGILANCE, match, jnp.zeros_like(match))
    out_ref[...] = thr
    mx = jnp.max(thr, axis=1, keepdims=True)
    lane = jax.lax.broadcasted_iota(jnp.int32, thr.shape, 1)
    idx = jnp.min(jnp.where(thr == mx, lane, jnp.int32(2**30)),
                  axis=1, keepdims=True)
    idx_ref[...] = idx


def kernel(x, categories):
    n, d = x.shape
    k = categories.shape[0]
    ct = categories.T

    scores, idx = pl.pallas_call(
        _fuzzy_art_block,
        grid=(n // BLOCK_ROWS,),
        in_specs=[
            pl.BlockSpec((BLOCK_ROWS, d), lambda i: (i, 0)),
            pl.BlockSpec((d, k), lambda i: (0, 0)),
        ],
        out_specs=[
            pl.BlockSpec((BLOCK_ROWS, k), lambda i: (i, 0)),
            pl.BlockSpec((BLOCK_ROWS, 1), lambda i: (i, 0)),
        ],
        out_shape=[
            jax.ShapeDtypeStruct((n, k), jnp.float32),
            jax.ShapeDtypeStruct((n, 1), jnp.int32),
        ],
    )(x, ct)
    return scores, idx.reshape(n)


# speedup vs baseline: 11.9553x; 1.6566x over previous
import jax
import jax.numpy as jnp
from jax.experimental import pallas as pl

VIGILANCE = 0.5
BLOCK_ROWS = 64
K_CHUNK = 2048
D_SUB = 8


def _fuzzy_art_block(x_ref, ct_ref, out_ref, idx_ref):
    bi = x_ref.shape[0]
    d = ct_ref.shape[0]
    k = ct_ref.shape[1]
    n_rows = d // D_SUB

    def row_body(s, carry):
        xrow = x_ref[pl.ds(s, 1), :]
        xcol = xrow.reshape(d, 1)
        for kc in range(0, k, K_CHUNK):
            sl = pl.ds(kc, K_CHUNK)
            acc = jnp.minimum(ct_ref[pl.ds(0, D_SUB), sl], xcol[0:D_SUB])
            for r in range(1, n_rows):
                acc = acc + jnp.minimum(ct_ref[pl.ds(r * D_SUB, D_SUB), sl],
                                        xcol[r * D_SUB:(r + 1) * D_SUB])
            out_ref[pl.ds(s, 1), sl] = jnp.sum(acc, axis=0, keepdims=True)
        return carry

    jax.lax.fori_loop(0, bi, row_body, 0)

    xsum = jnp.sum(x_ref[...], axis=1, keepdims=True)
    match = out_ref[...] / xsum
    thr = jnp.where(match >= VIGILANCE, match, jnp.zeros_like(match))
    out_ref[...] = thr
    mx = jnp.max(thr, axis=1, keepdims=True)
    lane = jax.lax.broadcasted_iota(jnp.int32, thr.shape, 1)
    idx = jnp.min(jnp.where(thr == mx, lane, jnp.int32(2**30)),
                  axis=1, keepdims=True)
    idx_ref[...] = idx


def kernel(x, categories):
    n, d = x.shape
    k = categories.shape[0]
    ct = categories.T

    scores, idx = pl.pallas_call(
        _fuzzy_art_block,
        grid=(n // BLOCK_ROWS,),
        in_specs=[
            pl.BlockSpec((BLOCK_ROWS, d), lambda i: (i, 0)),
            pl.BlockSpec((d, k), lambda i: (0, 0)),
        ],
        out_specs=[
            pl.BlockSpec((BLOCK_ROWS, k), lambda i: (i, 0)),
            pl.BlockSpec((BLOCK_ROWS, 1), lambda i: (i, 0)),
        ],
        out_shape=[
            jax.ShapeDtypeStruct((n, k), jnp.float32),
            jax.ShapeDtypeStruct((n, 1), jnp.int32),
        ],
    )(x, ct)
    return scores, idx.reshape(n)


# speedup vs baseline: 12.5757x; 1.0519x over previous
import jax
import jax.numpy as jnp
from jax.experimental import pallas as pl

VIGILANCE = 0.5
BLOCK_ROWS = 64
K_CHUNK = 1024
D_SUB = 8


def _fuzzy_art_block(x_ref, ct_ref, out_ref, idx_ref):
    bi = x_ref.shape[0]
    d = ct_ref.shape[0]
    k = ct_ref.shape[1]
    n_rows = d // D_SUB

    def pair_body(j, carry):
        s0 = 2 * j
        xcol_a = x_ref[pl.ds(s0, 1), :].reshape(d, 1)
        xcol_b = x_ref[pl.ds(s0 + 1, 1), :].reshape(d, 1)
        for kc in range(0, k, K_CHUNK):
            sl = pl.ds(kc, K_CHUNK)
            cr = ct_ref[pl.ds(0, D_SUB), sl]
            acc_a = jnp.minimum(cr, xcol_a[0:D_SUB])
            acc_b = jnp.minimum(cr, xcol_b[0:D_SUB])
            for r in range(1, n_rows):
                cr = ct_ref[pl.ds(r * D_SUB, D_SUB), sl]
                xs = slice(r * D_SUB, (r + 1) * D_SUB)
                acc_a = acc_a + jnp.minimum(cr, xcol_a[xs])
                acc_b = acc_b + jnp.minimum(cr, xcol_b[xs])
            out_ref[pl.ds(s0, 1), sl] = jnp.sum(acc_a, axis=0, keepdims=True)
            out_ref[pl.ds(s0 + 1, 1), sl] = jnp.sum(acc_b, axis=0,
                                                    keepdims=True)
        return carry

    jax.lax.fori_loop(0, bi // 2, pair_body, 0)

    xsum = jnp.sum(x_ref[...], axis=1, keepdims=True)
    match = out_ref[...] / xsum
    thr = jnp.where(match >= VIGILANCE, match, jnp.zeros_like(match))
    out_ref[...] = thr
    mx = jnp.max(thr, axis=1, keepdims=True)
    lane = jax.lax.broadcasted_iota(jnp.int32, thr.shape, 1)
    idx = jnp.min(jnp.where(thr == mx, lane, jnp.int32(2**30)),
                  axis=1, keepdims=True)
    idx_ref[...] = idx


def kernel(x, categories):
    n, d = x.shape
    k = categories.shape[0]
    ct = categories.T

    scores, idx = pl.pallas_call(
        _fuzzy_art_block,
        grid=(n // BLOCK_ROWS,),
        in_specs=[
            pl.BlockSpec((BLOCK_ROWS, d), lambda i: (i, 0)),
            pl.BlockSpec((d, k), lambda i: (0, 0)),
        ],
        out_specs=[
            pl.BlockSpec((BLOCK_ROWS, k), lambda i: (i, 0)),
            pl.BlockSpec((BLOCK_ROWS, 1), lambda i: (i, 0)),
        ],
        out_shape=[
            jax.ShapeDtypeStruct((n, k), jnp.float32),
            jax.ShapeDtypeStruct((n, 1), jnp.int32),
        ],
    )(x, ct)
    return scores, idx.reshape(n)


# speedup vs baseline: 12.8758x; 1.0239x over previous
import jax
import jax.numpy as jnp
from jax.experimental import pallas as pl

VIGILANCE = 0.5
BLOCK_ROWS = 64
K_CHUNK = 1024
D_SUB = 8


def _fuzzy_art_block(x_ref, ct_ref, out_ref, idx_ref):
    bi = x_ref.shape[0]
    d = ct_ref.shape[0]
    k = ct_ref.shape[1]
    n_rows = d // D_SUB

    def pair_body(j, carry):
        s0 = 2 * j
        xcol_a = x_ref[pl.ds(s0, 1), :].reshape(d, 1)
        xcol_b = x_ref[pl.ds(s0 + 1, 1), :].reshape(d, 1)
        for kc in range(0, k, K_CHUNK):
            sl = pl.ds(kc, K_CHUNK)
            cr = ct_ref[pl.ds(0, D_SUB), sl]
            acc_a = jnp.minimum(cr, xcol_a[0:D_SUB])
            acc_b = jnp.minimum(cr, xcol_b[0:D_SUB])
            for r in range(1, n_rows):
                cr = ct_ref[pl.ds(r * D_SUB, D_SUB), sl]
                xs = slice(r * D_SUB, (r + 1) * D_SUB)
                acc_a = acc_a + jnp.minimum(cr, xcol_a[xs])
                acc_b = acc_b + jnp.minimum(cr, xcol_b[xs])
            out_ref[pl.ds(s0, 1), sl] = jnp.sum(acc_a, axis=0, keepdims=True)
            out_ref[pl.ds(s0 + 1, 1), sl] = jnp.sum(acc_b, axis=0,
                                                    keepdims=True)
        return carry

    jax.lax.fori_loop(0, bi // 2, pair_body, 0, unroll=2)

    xsum = jnp.sum(x_ref[...], axis=1, keepdims=True)
    match = out_ref[...] / xsum
    thr = jnp.where(match >= VIGILANCE, match, jnp.zeros_like(match))
    out_ref[...] = thr
    mx = jnp.max(thr, axis=1, keepdims=True)
    lane = jax.lax.broadcasted_iota(jnp.int32, thr.shape, 1)
    idx = jnp.min(jnp.where(thr == mx, lane, jnp.int32(2**30)),
                  axis=1, keepdims=True)
    idx_ref[...] = idx


def kernel(x, categories):
    n, d = x.shape
    k = categories.shape[0]
    ct = categories.T

    scores, idx = pl.pallas_call(
        _fuzzy_art_block,
        grid=(n // BLOCK_ROWS,),
        in_specs=[
            pl.BlockSpec((BLOCK_ROWS, d), lambda i: (i, 0)),
            pl.BlockSpec((d, k), lambda i: (0, 0)),
        ],
        out_specs=[
            pl.BlockSpec((BLOCK_ROWS, k), lambda i: (i, 0)),
            pl.BlockSpec((BLOCK_ROWS, 1), lambda i: (i, 0)),
        ],
        out_shape=[
            jax.ShapeDtypeStruct((n, k), jnp.float32),
            jax.ShapeDtypeStruct((n, 1), jnp.int32),
        ],
    )(x, ct)
    return scores, idx.reshape(n)


# speedup vs baseline: 12.9064x; 1.0024x over previous
import jax
import jax.numpy as jnp
from jax.experimental import pallas as pl

VIGILANCE = 0.5
BLOCK_ROWS = 64
K_CHUNK = 1024
D_SUB = 8


def _fuzzy_art_block(x_ref, ct_ref, out_ref, idx_ref):
    bi = x_ref.shape[0]
    d = ct_ref.shape[0]
    k = ct_ref.shape[1]
    n_rows = d // D_SUB

    n_unroll = 4

    def group_body(j, carry):
        s0 = n_unroll * j
        xcols = [x_ref[pl.ds(s0 + u, 1), :].reshape(d, 1)
                 for u in range(n_unroll)]
        for kc in range(0, k, K_CHUNK):
            sl = pl.ds(kc, K_CHUNK)
            cr = ct_ref[pl.ds(0, D_SUB), sl]
            accs = [jnp.minimum(cr, xc[0:D_SUB]) for xc in xcols]
            for r in range(1, n_rows):
                cr = ct_ref[pl.ds(r * D_SUB, D_SUB), sl]
                xs = slice(r * D_SUB, (r + 1) * D_SUB)
                accs = [a + jnp.minimum(cr, xc[xs])
                        for a, xc in zip(accs, xcols)]
            for u in range(n_unroll):
                out_ref[pl.ds(s0 + u, 1), sl] = jnp.sum(accs[u], axis=0,
                                                        keepdims=True)
        return carry

    jax.lax.fori_loop(0, bi // n_unroll, group_body, 0)

    xsum = jnp.sum(x_ref[...], axis=1, keepdims=True)
    match = out_ref[...] / xsum
    thr = jnp.where(match >= VIGILANCE, match, jnp.zeros_like(match))
    out_ref[...] = thr
    mx = jnp.max(thr, axis=1, keepdims=True)
    lane = jax.lax.broadcasted_iota(jnp.int32, thr.shape, 1)
    idx = jnp.min(jnp.where(thr == mx, lane, jnp.int32(2**30)),
                  axis=1, keepdims=True)
    idx_ref[...] = idx


def kernel(x, categories):
    n, d = x.shape
    k = categories.shape[0]
    ct = categories.T

    scores, idx = pl.pallas_call(
        _fuzzy_art_block,
        grid=(n // BLOCK_ROWS,),
        in_specs=[
            pl.BlockSpec((BLOCK_ROWS, d), lambda i: (i, 0)),
            pl.BlockSpec((d, k), lambda i: (0, 0)),
        ],
        out_specs=[
            pl.BlockSpec((BLOCK_ROWS, k), lambda i: (i, 0)),
            pl.BlockSpec((BLOCK_ROWS, 1), lambda i: (i, 0)),
        ],
        out_shape=[
            jax.ShapeDtypeStruct((n, k), jnp.float32),
            jax.ShapeDtypeStruct((n, 1), jnp.int32),
        ],
    )(x, ct)
    return scores, idx.reshape(n)


# speedup vs baseline: 13.2611x; 1.0275x over previous
import jax
import jax.numpy as jnp
from jax.experimental import pallas as pl

VIGILANCE = 0.5
BLOCK_ROWS = 64
K_CHUNK = 512
D_SUB = 8


def _fuzzy_art_block(x_ref, ct_ref, out_ref, idx_ref):
    bi = x_ref.shape[0]
    d = ct_ref.shape[0]
    k = ct_ref.shape[1]
    n_rows = d // D_SUB

    n_unroll = 8

    def group_body(j, carry):
        s0 = n_unroll * j
        xcols = [x_ref[pl.ds(s0 + u, 1), :].reshape(d, 1)
                 for u in range(n_unroll)]
        for kc in range(0, k, K_CHUNK):
            sl = pl.ds(kc, K_CHUNK)
            cr = ct_ref[pl.ds(0, D_SUB), sl]
            accs = [jnp.minimum(cr, xc[0:D_SUB]) for xc in xcols]
            for r in range(1, n_rows):
                cr = ct_ref[pl.ds(r * D_SUB, D_SUB), sl]
                xs = slice(r * D_SUB, (r + 1) * D_SUB)
                accs = [a + jnp.minimum(cr, xc[xs])
                        for a, xc in zip(accs, xcols)]
            rows = jnp.concatenate(
                [jnp.sum(a, axis=0, keepdims=True) for a in accs], axis=0)
            out_ref[pl.ds(s0, n_unroll), sl] = rows
        return carry

    jax.lax.fori_loop(0, bi // n_unroll, group_body, 0)

    xsum = jnp.sum(x_ref[...], axis=1, keepdims=True)
    match = out_ref[...] / xsum
    thr = jnp.where(match >= VIGILANCE, match, jnp.zeros_like(match))
    out_ref[...] = thr
    mx = jnp.max(thr, axis=1, keepdims=True)
    lane = jax.lax.broadcasted_iota(jnp.int32, thr.shape, 1)
    idx = jnp.min(jnp.where(thr == mx, lane, jnp.int32(2**30)),
                  axis=1, keepdims=True)
    idx_ref[...] = idx


def kernel(x, categories):
    n, d = x.shape
    k = categories.shape[0]
    ct = categories.T

    scores, idx = pl.pallas_call(
        _fuzzy_art_block,
        grid=(n // BLOCK_ROWS,),
        in_specs=[
            pl.BlockSpec((BLOCK_ROWS, d), lambda i: (i, 0)),
            pl.BlockSpec((d, k), lambda i: (0, 0)),
        ],
        out_specs=[
            pl.BlockSpec((BLOCK_ROWS, k), lambda i: (i, 0)),
            pl.BlockSpec((BLOCK_ROWS, 1), lambda i: (i, 0)),
        ],
        out_shape=[
            jax.ShapeDtypeStruct((n, k), jnp.float32),
            jax.ShapeDtypeStruct((n, 1), jnp.int32),
        ],
    )(x, ct)
    return scores, idx.reshape(n)


# speedup vs baseline: 13.3025x; 1.0031x over previous
import jax
import jax.numpy as jnp
from jax.experimental import pallas as pl

VIGILANCE = 0.5
BLOCK_ROWS = 128
K_CHUNK = 512
D_SUB = 8


def _fuzzy_art_block(x_ref, ct_ref, out_ref, idx_ref):
    bi = x_ref.shape[0]
    d = ct_ref.shape[0]
    k = ct_ref.shape[1]
    n_rows = d // D_SUB

    n_unroll = 8

    def group_body(j, carry):
        s0 = n_unroll * j
        xcols = [x_ref[pl.ds(s0 + u, 1), :].reshape(d, 1)
                 for u in range(n_unroll)]
        for kc in range(0, k, K_CHUNK):
            sl = pl.ds(kc, K_CHUNK)
            cr = ct_ref[pl.ds(0, D_SUB), sl]
            accs = [jnp.minimum(cr, xc[0:D_SUB]) for xc in xcols]
            for r in range(1, n_rows):
                cr = ct_ref[pl.ds(r * D_SUB, D_SUB), sl]
                xs = slice(r * D_SUB, (r + 1) * D_SUB)
                accs = [a + jnp.minimum(cr, xc[xs])
                        for a, xc in zip(accs, xcols)]
            rows = jnp.concatenate(
                [jnp.sum(a, axis=0, keepdims=True) for a in accs], axis=0)
            out_ref[pl.ds(s0, n_unroll), sl] = rows
        return carry

    jax.lax.fori_loop(0, bi // n_unroll, group_body, 0)

    xsum = jnp.sum(x_ref[...], axis=1, keepdims=True)
    match = out_ref[...] / xsum
    thr = jnp.where(match >= VIGILANCE, match, jnp.zeros_like(match))
    out_ref[...] = thr
    mx = jnp.max(thr, axis=1, keepdims=True)
    lane = jax.lax.broadcasted_iota(jnp.int32, thr.shape, 1)
    idx = jnp.min(jnp.where(thr == mx, lane, jnp.int32(2**30)),
                  axis=1, keepdims=True)
    idx_ref[...] = idx


def kernel(x, categories):
    n, d = x.shape
    k = categories.shape[0]
    ct = categories.T

    scores, idx = pl.pallas_call(
        _fuzzy_art_block,
        grid=(n // BLOCK_ROWS,),
        in_specs=[
            pl.BlockSpec((BLOCK_ROWS, d), lambda i: (i, 0)),
            pl.BlockSpec((d, k), lambda i: (0, 0)),
        ],
        out_specs=[
            pl.BlockSpec((BLOCK_ROWS, k), lambda i: (i, 0)),
            pl.BlockSpec((BLOCK_ROWS, 1), lambda i: (i, 0)),
        ],
        out_shape=[
            jax.ShapeDtypeStruct((n, k), jnp.float32),
            jax.ShapeDtypeStruct((n, 1), jnp.int32),
        ],
    )(x, ct)
    return scores, idx.reshape(n)


# speedup vs baseline: 13.7991x; 1.0373x over previous
import jax
import jax.numpy as jnp
from jax.experimental import pallas as pl
from jax.experimental.pallas import tpu as pltpu

VIGILANCE = 0.5
BLOCK_ROWS = 128
K_CHUNK = 512
D_SUB = 8


def _fuzzy_art_block(x_ref, c_ref, out_ref, idx_ref, ct_ref):
    bi = x_ref.shape[0]
    kk, d = c_ref.shape
    k = ct_ref.shape[1]
    n_rows = d // D_SUB

    @pl.when(pl.program_id(0) == 0)
    def _transpose_codebook():
        for cchunk in range(0, kk, 512):
            ct_ref[:, pl.ds(cchunk, 512)] = c_ref[pl.ds(cchunk, 512), :].T

    n_unroll = 8

    def group_body(j, carry):
        s0 = n_unroll * j
        xcols = [x_ref[pl.ds(s0 + u, 1), :].reshape(d, 1)
                 for u in range(n_unroll)]
        for kc in range(0, k, K_CHUNK):
            sl = pl.ds(kc, K_CHUNK)
            cr = ct_ref[pl.ds(0, D_SUB), sl]
            accs = [jnp.minimum(cr, xc[0:D_SUB]) for xc in xcols]
            for r in range(1, n_rows):
                cr = ct_ref[pl.ds(r * D_SUB, D_SUB), sl]
                xs = slice(r * D_SUB, (r + 1) * D_SUB)
                accs = [a + jnp.minimum(cr, xc[xs])
                        for a, xc in zip(accs, xcols)]
            rows = jnp.concatenate(
                [jnp.sum(a, axis=0, keepdims=True) for a in accs], axis=0)
            out_ref[pl.ds(s0, n_unroll), sl] = rows
        return carry

    jax.lax.fori_loop(0, bi // n_unroll, group_body, 0)

    xsum = jnp.sum(x_ref[...], axis=1, keepdims=True)
    match = out_ref[...] / xsum
    thr = jnp.where(match >= VIGILANCE, match, jnp.zeros_like(match))
    out_ref[...] = thr
    mx = jnp.max(thr, axis=1, keepdims=True)
    lane = jax.lax.broadcasted_iota(jnp.int32, thr.shape, 1)
    idx = jnp.min(jnp.where(thr == mx, lane, jnp.int32(2**30)),
                  axis=1, keepdims=True)
    idx_ref[...] = idx


def kernel(x, categories):
    n, d = x.shape
    k = categories.shape[0]
    scores, idx = pl.pallas_call(
        _fuzzy_art_block,
        grid=(n // BLOCK_ROWS,),
        in_specs=[
            pl.BlockSpec((BLOCK_ROWS, d), lambda i: (i, 0)),
            pl.BlockSpec((k, d), lambda i: (0, 0)),
        ],
        out_specs=[
            pl.BlockSpec((BLOCK_ROWS, k), lambda i: (i, 0)),
            pl.BlockSpec((BLOCK_ROWS, 1), lambda i: (i, 0)),
        ],
        out_shape=[
            jax.ShapeDtypeStruct((n, k), jnp.float32),
            jax.ShapeDtypeStruct((n, 1), jnp.int32),
        ],
        scratch_shapes=[pltpu.VMEM((d, k), jnp.float32)],
    )(x, categories)
    return scores, idx.reshape(n)
